# baseline (speedup 1.0000x reference)
import functools

import jax
import jax.numpy as jnp
from jax import lax
from jax.experimental import pallas as pl
from jax.experimental.pallas import tpu as pltpu
from jax.experimental.pallas import tpu_sc as plsc

N = 10000
D = 128
H = 8
E = 320000
W = 288
NS = 16
CHUNK = 64
CH = 320
IBLK = 32
NBUF = 2
NBLK = CH // IBLK
EPT = CH * CHUNK
E_PAD = NS * EPT
RPT = 625
BLK = 400
NT = N + BLK
INV_H = 1.0 / H


def _table_body(x_ref, wv_ref, bv_ref, wm_ref, bm_ref, out_ref):
    i = pl.program_id(0)
    x = x_ref[...]
    v = jnp.dot(x, wv_ref[...], preferred_element_type=jnp.float32) + bv_ref[...]
    vm = jnp.dot(v, wm_ref[...], preferred_element_type=jnp.float32) + bm_ref[...]
    col = lax.broadcasted_iota(jnp.int32, (BLK, W - D), 1)
    one_pad = jnp.where(col == 0, 1.0, 0.0)
    row = jnp.concatenate([vm * INV_H, one_pad], axis=1)
    out_ref[...] = row * jnp.where(i < N // BLK, 1.0, 0.0)


def _vm_table(x, Wv, bv, Wm, bm):
    return pl.pallas_call(
        _table_body,
        grid=(N // BLK + 1,),
        in_specs=[
            pl.BlockSpec((BLK, D), lambda i: (jnp.minimum(i, N // BLK - 1), 0)),
            pl.BlockSpec((D, D), lambda i: (0, 0)),
            pl.BlockSpec((1, D), lambda i: (0, 0)),
            pl.BlockSpec((D, D), lambda i: (0, 0)),
            pl.BlockSpec((1, D), lambda i: (0, 0)),
        ],
        out_specs=pl.BlockSpec((BLK, W), lambda i: (i, 0)),
        out_shape=jax.ShapeDtypeStruct((NT, W), jnp.float32),
    )(x, Wv, bv.reshape(1, D), Wm, bm.reshape(1, D))


def _out_body(acc_ref, x_ref, wo_ref, bo_ref, y_ref):
    a = acc_ref[...]
    cnt = jnp.maximum(a[:, D:D + 1], 1.0)
    norm = a[:, :D] / cnt
    y = jnp.dot(norm, wo_ref[...], preferred_element_type=jnp.float32)
    y_ref[...] = jnp.maximum(y + bo_ref[...] + x_ref[...], 0.0)


def _project_out(acc, x, Wo, bo):
    return pl.pallas_call(
        _out_body,
        grid=(N // BLK,),
        in_specs=[
            pl.BlockSpec((BLK, 144), lambda i: (i, 0)),
            pl.BlockSpec((BLK, D), lambda i: (i, 0)),
            pl.BlockSpec((D, D), lambda i: (0, 0)),
            pl.BlockSpec((1, D), lambda i: (0, 0)),
        ],
        out_specs=pl.BlockSpec((BLK, D), lambda i: (i, 0)),
        out_shape=jax.ShapeDtypeStruct((N, D), jnp.float32),
    )(acc, x, Wo, bo.reshape(1, D))


def _sc_body(table_p, sidx_p, didx_p, table_r, sidx_r, didx_r, zeros_h,
             out_p, out_r,
             sidx_v, didx_v, rows_a, rows_b, rows_c, rows_d, acc,
             sem_a, sem_b, sem_c, sem_d):
    rows = (rows_a, rows_b, rows_c, rows_d)
    sems = (sem_a, sem_b, sem_c, sem_d)
    c = lax.axis_index("c")
    s = lax.axis_index("s")

    def run(table_h, sidx_h, didx_h, out_h):
        pltpu.sync_copy(zeros_h, acc.at[pl.ds(s * RPT, RPT)])
        plsc.subcore_barrier()

        @pl.loop(0, NBLK)
        def _(b):
            base = s * CH + b * IBLK
            pltpu.sync_copy(sidx_h.at[pl.ds(base, IBLK)], sidx_v)
            pltpu.sync_copy(didx_h.at[pl.ds(base, IBLK)], didx_v)

            for p in range(NBUF - 1):
                pltpu.async_copy(table_h.at[sidx_v.at[p]], rows[p], sems[p])

            @pl.loop(0, IBLK, step=NBUF)
            def _(j):
                for p in range(NBUF):
                    pltpu.make_async_copy(table_h.at[sidx_v.at[j + p]],
                                          rows[p], sems[p]).wait()
                    q = (p + NBUF - 1) % NBUF

                    @pl.when(j + p + NBUF - 1 < IBLK)
                    def _():
                        pltpu.async_copy(
                            table_h.at[sidx_v.at[j + p + NBUF - 1]],
                            rows[q], sems[q])


        plsc.subcore_barrier()
        pltpu.sync_copy(acc.at[pl.ds(s * RPT, RPT)], out_h.at[pl.ds(s * RPT, RPT)])

    @pl.when(c == 0)
    def _():
        run(table_p, sidx_p, didx_p, out_p)

    @pl.when(c == 1)
    def _():
        run(table_r, sidx_r, didx_r, out_r)


_sc_scatter = functools.partial(
    pl.kernel,
    out_type=[jax.ShapeDtypeStruct((N, 144), jnp.float32),
              jax.ShapeDtypeStruct((N, 144), jnp.float32)],
    mesh=plsc.VectorSubcoreMesh(core_axis_name="c", subcore_axis_name="s"),
    compiler_params=pltpu.CompilerParams(use_tc_tiling_on_sc=False),
    scratch_types=[
        pltpu.VMEM((IBLK, CHUNK), jnp.int32),
        pltpu.VMEM((IBLK, CHUNK), jnp.int32),
        pltpu.VMEM((CHUNK, W), jnp.float32),
        pltpu.VMEM((CHUNK, W), jnp.float32),
        pltpu.VMEM((CHUNK, W), jnp.float32),
        pltpu.VMEM((CHUNK, W), jnp.float32),
        pltpu.VMEM_SHARED((N, 144), jnp.float32),
        pltpu.SemaphoreType.DMA,
        pltpu.SemaphoreType.DMA,
        pltpu.SemaphoreType.DMA,
        pltpu.SemaphoreType.DMA,
    ],
)(_sc_body)


def _pad_idx(row, fill):
    pad = jnp.full((E_PAD - E,), fill, jnp.int32)
    return jnp.concatenate([row, pad]).reshape(NS * CH, CHUNK)


def kernel(x_user, x_game, edge_index_played, edge_index_rev,
           Wk_u, bk_u, Wq_u, bq_u, Wv_u, bv_u, Wo_u, bo_u,
           Wk_g, bk_g, Wq_g, bq_g, Wv_g, bv_g, Wo_g, bo_g,
           Wa_p, ba_p, Wm_p, bm_p, Wa_r, ba_r, Wm_r, bm_r):
    table_p = _vm_table(x_user, Wv_u, bv_u, Wm_p, bm_p)
    table_r = _vm_table(x_game, Wv_g, bv_g, Wm_r, bm_r)

    sidx_p = _pad_idx(edge_index_played[0], N)
    didx_p = _pad_idx(edge_index_played[1], 0)
    sidx_r = _pad_idx(edge_index_rev[0], N)
    didx_r = _pad_idx(edge_index_rev[1], 0)

    zeros_h = jnp.zeros((RPT, 144), jnp.float32)
    acc_p, acc_r = _sc_scatter(table_p, sidx_p, didx_p,
                               table_r, sidx_r, didx_r, zeros_h)

    y_g = _project_out(acc_p, x_game, Wo_g, bo_g)
    y_u = _project_out(acc_r, x_user, Wo_u, bo_u)
    return (y_u, y_g)


# speedup vs baseline: 1.7254x; 1.7254x over previous
import functools

import jax
import jax.numpy as jnp
from jax import lax
from jax.experimental import pallas as pl
from jax.experimental.pallas import tpu as pltpu
from jax.experimental.pallas import tpu_sc as plsc

N = 10000
D = 128
H = 8
E = 320000
W = 144
NS = 16
CHUNK = 32
CH = 640
IBLK = 32
NBUF = 8
NBLK = CH // IBLK
EPT = CH * CHUNK
E_PAD = NS * EPT
RPT = 625
BLK = 400
NT = N + BLK
INV_H = 1.0 / H


def _table_body(x_ref, wv_ref, bv_ref, wm_ref, bm_ref, out_ref):
    i = pl.program_id(0)
    x = x_ref[...]
    v = jnp.dot(x, wv_ref[...], preferred_element_type=jnp.float32) + bv_ref[...]
    vm = jnp.dot(v, wm_ref[...], preferred_element_type=jnp.float32) + bm_ref[...]
    col = lax.broadcasted_iota(jnp.int32, (BLK, W - D), 1)
    one_pad = jnp.where(col == 0, 1.0, 0.0)
    row = jnp.concatenate([vm * INV_H, one_pad], axis=1)
    out_ref[...] = row * jnp.where(i < N // BLK, 1.0, 0.0)


def _vm_table(x, Wv, bv, Wm, bm):
    return pl.pallas_call(
        _table_body,
        grid=(N // BLK + 1,),
        in_specs=[
            pl.BlockSpec((BLK, D), lambda i: (jnp.minimum(i, N // BLK - 1), 0)),
            pl.BlockSpec((D, D), lambda i: (0, 0)),
            pl.BlockSpec((1, D), lambda i: (0, 0)),
            pl.BlockSpec((D, D), lambda i: (0, 0)),
            pl.BlockSpec((1, D), lambda i: (0, 0)),
        ],
        out_specs=pl.BlockSpec((BLK, W), lambda i: (i, 0)),
        out_shape=jax.ShapeDtypeStruct((NT, W), jnp.float32),
    )(x, Wv, bv.reshape(1, D), Wm, bm.reshape(1, D))


def _out_body(acc_ref, x_ref, wo_ref, bo_ref, y_ref):
    a = acc_ref[...]
    cnt = jnp.maximum(a[:, D:D + 1], 1.0)
    norm = a[:, :D] / cnt
    y = jnp.dot(norm, wo_ref[...], preferred_element_type=jnp.float32)
    y_ref[...] = jnp.maximum(y + bo_ref[...] + x_ref[...], 0.0)


def _project_out(acc, x, Wo, bo):
    return pl.pallas_call(
        _out_body,
        grid=(N // BLK,),
        in_specs=[
            pl.BlockSpec((BLK, W), lambda i: (i, 0)),
            pl.BlockSpec((BLK, D), lambda i: (i, 0)),
            pl.BlockSpec((D, D), lambda i: (0, 0)),
            pl.BlockSpec((1, D), lambda i: (0, 0)),
        ],
        out_specs=pl.BlockSpec((BLK, D), lambda i: (i, 0)),
        out_shape=jax.ShapeDtypeStruct((N, D), jnp.float32),
    )(acc, x, Wo, bo.reshape(1, D))


def _sc_body(table_p, sidx_p, didx_p, table_r, sidx_r, didx_r, zeros_h,
             out_p, out_r,
             sidx_v, didx_v, rows_a, rows_b, rows_c, rows_d, rows_e, rows_f,
             rows_g, rows_h2, acc,
             sem_a, sem_b, sem_c, sem_d, sem_e, sem_f, sem_g, sem_h2):
    rows = (rows_a, rows_b, rows_c, rows_d, rows_e, rows_f, rows_g, rows_h2)
    sems = (sem_a, sem_b, sem_c, sem_d, sem_e, sem_f, sem_g, sem_h2)
    c = lax.axis_index("c")
    s = lax.axis_index("s")

    def run(table_h, sidx_h, didx_h, out_h):
        pltpu.sync_copy(zeros_h, acc.at[pl.ds(s * RPT, RPT)])
        plsc.subcore_barrier()

        @pl.loop(0, NBLK)
        def _(b):
            base = s * CH + b * IBLK
            pltpu.sync_copy(sidx_h.at[pl.ds(base, IBLK)], sidx_v)
            pltpu.sync_copy(didx_h.at[pl.ds(base, IBLK)], didx_v)

            for p in range(NBUF - 1):
                pltpu.async_copy(table_h.at[sidx_v.at[p]], rows[p], sems[p])

            @pl.loop(0, IBLK, step=NBUF)
            def _(j):
                for p in range(NBUF):
                    pltpu.make_async_copy(table_h.at[sidx_v.at[j + p]],
                                          rows[p], sems[p]).wait()
                    q = (p + NBUF - 1) % NBUF

                    @pl.when(j + p + NBUF - 1 < IBLK)
                    def _():
                        pltpu.async_copy(
                            table_h.at[sidx_v.at[j + p + NBUF - 1]],
                            rows[q], sems[q])

                    pltpu.sync_copy(rows[p], acc.at[didx_v.at[j + p]],
                                    add=True)

        plsc.subcore_barrier()
        pltpu.sync_copy(acc.at[pl.ds(s * RPT, RPT)], out_h.at[pl.ds(s * RPT, RPT)])

    @pl.when(c == 0)
    def _():
        run(table_p, sidx_p, didx_p, out_p)

    @pl.when(c == 1)
    def _():
        run(table_r, sidx_r, didx_r, out_r)


_sc_scatter = functools.partial(
    pl.kernel,
    out_type=[jax.ShapeDtypeStruct((N, W), jnp.float32),
              jax.ShapeDtypeStruct((N, W), jnp.float32)],
    mesh=plsc.VectorSubcoreMesh(core_axis_name="c", subcore_axis_name="s"),
    compiler_params=pltpu.CompilerParams(use_tc_tiling_on_sc=False),
    scratch_types=[
        pltpu.VMEM((IBLK, CHUNK), jnp.int32),
        pltpu.VMEM((IBLK, CHUNK), jnp.int32),
        pltpu.VMEM((CHUNK, W), jnp.float32),
        pltpu.VMEM((CHUNK, W), jnp.float32),
        pltpu.VMEM((CHUNK, W), jnp.float32),
        pltpu.VMEM((CHUNK, W), jnp.float32),
        pltpu.VMEM((CHUNK, W), jnp.float32),
        pltpu.VMEM((CHUNK, W), jnp.float32),
        pltpu.VMEM((CHUNK, W), jnp.float32),
        pltpu.VMEM((CHUNK, W), jnp.float32),
        pltpu.VMEM_SHARED((N, W), jnp.float32),
        pltpu.SemaphoreType.DMA,
        pltpu.SemaphoreType.DMA,
        pltpu.SemaphoreType.DMA,
        pltpu.SemaphoreType.DMA,
        pltpu.SemaphoreType.DMA,
        pltpu.SemaphoreType.DMA,
        pltpu.SemaphoreType.DMA,
        pltpu.SemaphoreType.DMA,
    ],
)(_sc_body)


def _pad_idx(row, fill):
    pad = jnp.full((E_PAD - E,), fill, jnp.int32)
    return jnp.concatenate([row, pad]).reshape(NS * CH, CHUNK)


def kernel(x_user, x_game, edge_index_played, edge_index_rev,
           Wk_u, bk_u, Wq_u, bq_u, Wv_u, bv_u, Wo_u, bo_u,
           Wk_g, bk_g, Wq_g, bq_g, Wv_g, bv_g, Wo_g, bo_g,
           Wa_p, ba_p, Wm_p, bm_p, Wa_r, ba_r, Wm_r, bm_r):
    table_p = _vm_table(x_user, Wv_u, bv_u, Wm_p, bm_p)
    table_r = _vm_table(x_game, Wv_g, bv_g, Wm_r, bm_r)

    sidx_p = _pad_idx(edge_index_played[0], N)
    didx_p = _pad_idx(edge_index_played[1], 0)
    sidx_r = _pad_idx(edge_index_rev[0], N)
    didx_r = _pad_idx(edge_index_rev[1], 0)

    zeros_h = jnp.zeros((RPT, W), jnp.float32)
    acc_p, acc_r = _sc_scatter(table_p, sidx_p, didx_p,
                               table_r, sidx_r, didx_r, zeros_h)

    y_g = _project_out(acc_p, x_game, Wo_g, bo_g)
    y_u = _project_out(acc_r, x_user, Wo_u, bo_u)
    return (y_u, y_g)
